# baseline (speedup 1.0000x reference)
import functools

import jax
import jax.numpy as jnp
from jax import lax
from jax.experimental import pallas as pl
from jax.experimental.pallas import tpu as pltpu

_SMOOTHING = 0.1
_CONF = 1.0 - _SMOOTHING
_BC = 4096
_B = 1024
_C = 100000
_CALN = 98304
_NCB = _CALN // _BC
_TW = _C - _CALN


def _loss_kernel(x_ref, xt_ref, t_ref, o_ref, m_ref, s_ref, rs_ref, g_ref):
    j = pl.program_id(0)

    @pl.when(j == 0)
    def _init():
        m_ref[...] = jnp.full_like(m_ref, -jnp.inf)
        s_ref[...] = jnp.zeros_like(s_ref)
        rs_ref[...] = jnp.zeros_like(rs_ref)
        g_ref[...] = jnp.zeros_like(g_ref)

    x = x_ref[...]
    cols = lax.broadcasted_iota(jnp.int32, x.shape, 1)
    tloc = t_ref[...] - j * _BC
    g_ref[...] += jnp.sum(jnp.where(cols == tloc, x, 0.0), axis=1,
                          keepdims=True)

    def _update(xv):
        chunk_max = jnp.max(xv, axis=1, keepdims=True)
        m_old = m_ref[...]
        m_new = jnp.maximum(m_old, chunk_max)
        s_ref[...] = s_ref[...] * jnp.exp(m_old - m_new) + jnp.sum(
            jnp.exp(xv - m_new), axis=1, keepdims=True)
        m_ref[...] = m_new
        rs_ref[...] += jnp.sum(xv, axis=1, keepdims=True)

    _update(x)

    @pl.when(j == _NCB - 1)
    def _last():
        xt = xt_ref[...]
        tcols = lax.broadcasted_iota(jnp.int32, xt.shape, 1)
        ttloc = t_ref[...] - _CALN
        g_ref[...] += jnp.sum(jnp.where(tcols == ttloc, xt, 0.0), axis=1,
                              keepdims=True)
        _update(xt)
        eps = _SMOOTHING / (_C - 1)
        lse = m_ref[...] + jnp.log(s_ref[...])
        rowloss = (-eps * (rs_ref[...] - _C * lse)
                   - (_CONF - eps) * (g_ref[...] - lse))
        o_ref[...] = (jnp.sum(rowloss) / _B).reshape(1, 1)


def kernel(pred, target):
    t2 = target.reshape(_B, 1).astype(jnp.int32)
    xt = lax.slice(pred, (0, _CALN), (_B, _C))
    out = pl.pallas_call(
        _loss_kernel,
        grid=(_NCB,),
        in_specs=[
            pl.BlockSpec((_B, _BC), lambda j: (0, j)),
            pl.BlockSpec((_B, _TW), lambda j: (0, 0)),
            pl.BlockSpec((_B, 1), lambda j: (0, 0)),
        ],
        out_specs=pl.BlockSpec((1, 1), lambda j: (0, 0)),
        out_shape=jax.ShapeDtypeStruct((1, 1), jnp.float32),
        scratch_shapes=[
            pltpu.VMEM((_B, 1), jnp.float32),
            pltpu.VMEM((_B, 1), jnp.float32),
            pltpu.VMEM((_B, 1), jnp.float32),
            pltpu.VMEM((_B, 1), jnp.float32),
        ],
        compiler_params=pltpu.CompilerParams(
            dimension_semantics=("arbitrary",)),
    )(pred, xt, t2)
    return out[0, 0]


# speedup vs baseline: 1.0003x; 1.0003x over previous
import jax
import jax.numpy as jnp
from jax import lax
from jax.experimental import pallas as pl
from jax.experimental.pallas import tpu as pltpu

_SMOOTHING = 0.1
_CONF = 1.0 - _SMOOTHING
_BC = 4096
_B = 1024
_C = 100000
_CALN = 98304
_NCB = _CALN // _BC
_TW = _C - _CALN


def _loss_kernel(pred_hbm, t_ref, o_ref, a0, a1, tbuf,
                 m_ref, s_ref, rs_ref, g_ref, s0, s1, st):
    pltpu.make_async_copy(
        pred_hbm.at[:, pl.ds(_CALN, _TW)], tbuf, st).start()
    bufs = (a0, a1)
    sems = (s0, s1)
    pltpu.make_async_copy(
        pred_hbm.at[:, pl.ds(0, _BC)], a0, s0).start()

    m_ref[...] = jnp.full_like(m_ref, -jnp.inf)
    s_ref[...] = jnp.zeros_like(s_ref)
    rs_ref[...] = jnp.zeros_like(rs_ref)
    g_ref[...] = jnp.zeros_like(g_ref)

    def _update(xv):
        chunk_max = jnp.max(xv, axis=1, keepdims=True)
        m_old = m_ref[...]
        m_new = jnp.maximum(m_old, chunk_max)
        s_ref[...] = s_ref[...] * jnp.exp(m_old - m_new) + jnp.sum(
            jnp.exp(xv - m_new), axis=1, keepdims=True)
        m_ref[...] = m_new
        rs_ref[...] += jnp.sum(xv, axis=1, keepdims=True)

    for j in range(_NCB):
        cur, csem = bufs[j % 2], sems[j % 2]
        pltpu.make_async_copy(
            pred_hbm.at[:, pl.ds(j * _BC, _BC)], cur, csem).wait()
        if j + 1 < _NCB:
            pltpu.make_async_copy(
                pred_hbm.at[:, pl.ds((j + 1) * _BC, _BC)],
                bufs[(j + 1) % 2], sems[(j + 1) % 2]).start()
        x = cur[...]
        cols = lax.broadcasted_iota(jnp.int32, x.shape, 1)
        tloc = t_ref[...] - j * _BC
        g_ref[...] += jnp.sum(jnp.where(cols == tloc, x, 0.0), axis=1,
                              keepdims=True)
        _update(x)

    pltpu.make_async_copy(
        pred_hbm.at[:, pl.ds(_CALN, _TW)], tbuf, st).wait()
    xt = tbuf[...]
    tcols = lax.broadcasted_iota(jnp.int32, xt.shape, 1)
    ttloc = t_ref[...] - _CALN
    g_ref[...] += jnp.sum(jnp.where(tcols == ttloc, xt, 0.0), axis=1,
                          keepdims=True)
    _update(xt)
    eps = _SMOOTHING / (_C - 1)
    lse = m_ref[...] + jnp.log(s_ref[...])
    rowloss = (-eps * (rs_ref[...] - _C * lse)
               - (_CONF - eps) * (g_ref[...] - lse))
    o_ref[...] = (jnp.sum(rowloss) / _B).reshape(1, 1)


def kernel(pred, target):
    t2 = target.reshape(_B, 1).astype(jnp.int32)
    out = pl.pallas_call(
        _loss_kernel,
        in_specs=[
            pl.BlockSpec(memory_space=pl.ANY),
            pl.BlockSpec((_B, 1), lambda: (0, 0)),
        ],
        out_specs=pl.BlockSpec((1, 1), lambda: (0, 0)),
        out_shape=jax.ShapeDtypeStruct((1, 1), jnp.float32),
        scratch_shapes=[
            pltpu.VMEM((_B, _BC), jnp.float32),
            pltpu.VMEM((_B, _BC), jnp.float32),
            pltpu.VMEM((_B, _TW), jnp.float32),
            pltpu.VMEM((_B, 1), jnp.float32),
            pltpu.VMEM((_B, 1), jnp.float32),
            pltpu.VMEM((_B, 1), jnp.float32),
            pltpu.VMEM((_B, 1), jnp.float32),
            pltpu.SemaphoreType.DMA,
            pltpu.SemaphoreType.DMA,
            pltpu.SemaphoreType.DMA,
        ],
    )(pred, t2)
    return out[0, 0]


# speedup vs baseline: 1.0095x; 1.0092x over previous
import functools

import jax
import jax.numpy as jnp
from jax import lax
from jax.experimental import pallas as pl
from jax.experimental.pallas import tpu as pltpu
from jax.experimental.pallas import tpu_sc as plsc

_SMOOTHING = 0.1
_CONF = 1.0 - _SMOOTHING

_B = 1024
_C = 100000
_BC = 4096
_C0 = 67200
_C1 = 99968
_W = _C1 - _C0
_TAIL = _C - _C1
_NTILES = 32
_RPT = _B // _NTILES
_LANES = 16
_GR = 8
_WC = 4096
_NG = _RPT // _GR
_NK = _W // _WC


def _tc_partial_kernel(x_ref, t_ref, m_o, s_o, rs_o, g_o,
                       m_ref, s_ref, rs_ref, g_ref, *, ncb):
    j = pl.program_id(0)

    @pl.when(j == 0)
    def _init():
        m_ref[...] = jnp.full_like(m_ref, -jnp.inf)
        s_ref[...] = jnp.zeros_like(s_ref)
        rs_ref[...] = jnp.zeros_like(rs_ref)
        g_ref[...] = jnp.zeros_like(g_ref)

    x = x_ref[...]
    cols = lax.broadcasted_iota(jnp.int32, x.shape, 1)
    tloc = t_ref[...] - j * _BC
    g_ref[...] += jnp.sum(jnp.where(cols == tloc, x, 0.0), axis=1,
                          keepdims=True)

    def _update(xm, xs):
        chunk_max = jnp.max(xm, axis=1, keepdims=True)
        m_old = m_ref[...]
        m_new = jnp.maximum(m_old, chunk_max)
        s_ref[...] = s_ref[...] * jnp.exp(m_old - m_new) + jnp.sum(
            jnp.exp(xm - m_new), axis=1, keepdims=True)
        m_ref[...] = m_new
        rs_ref[...] += jnp.sum(xs, axis=1, keepdims=True)

    @pl.when(j < ncb - 1)
    def _fast():
        _update(x, x)

    @pl.when(j == ncb - 1)
    def _last():
        mask = cols < (_C0 - (ncb - 1) * _BC)
        _update(jnp.where(mask, x, -jnp.inf), jnp.where(mask, x, 0.0))
        m_o[...] = m_ref[...]
        s_o[...] = s_ref[...]
        rs_o[...] = rs_ref[...]
        g_o[...] = g_ref[...]


def _tc_partial(pred, t2):
    ncb = pl.cdiv(_C0, _BC)
    one = jax.ShapeDtypeStruct((_B, 1), jnp.float32)
    return pl.pallas_call(
        functools.partial(_tc_partial_kernel, ncb=ncb),
        grid=(ncb,),
        in_specs=[
            pl.BlockSpec((_B, _BC), lambda j: (0, j)),
            pl.BlockSpec((_B, 1), lambda j: (0, 0)),
        ],
        out_specs=[pl.BlockSpec((_B, 1), lambda j: (0, 0))] * 4,
        out_shape=[one, one, one, one],
        scratch_shapes=[pltpu.VMEM((_B, 1), jnp.float32)] * 4,
        compiler_params=pltpu.CompilerParams(
            dimension_semantics=("arbitrary",)),
    )(pred, t2)


def _sc_body(pred_hbm, tgt_hbm, m_out, s_out, rs_out, g_out,
             vb0, vb1, st_m, st_s, st_rs, st_g, tl_v,
             sem0, sem1):
    wid = lax.axis_index("s") * 2 + lax.axis_index("c")
    r0 = wid * _RPT

    pltpu.sync_copy(tgt_hbm.at[pl.ds(r0, _RPT), :], tl_v)

    neg_inf = jnp.full((_LANES,), -jnp.inf, jnp.float32)
    zero = jnp.zeros((_LANES,), jnp.float32)
    lane_iota = lax.iota(jnp.int32, _LANES)

    def _init_rows(r, carry):
        st_m[r] = neg_inf
        st_s[r] = zero
        st_rs[r] = zero
        st_g[r] = zero
        return carry

    lax.fori_loop(0, _RPT, _init_rows, 0)

    bufs = (vb0, vb1)
    sems = (sem0, sem1)
    handles = [None, None]

    def _start(q, slot):
        g, k = divmod(q, _NK)
        handles[slot] = pltpu.async_copy(
            pred_hbm.at[pl.ds(r0 + g * _GR, _GR),
                        pl.ds(_C0 + k * _WC, _WC)],
            bufs[slot], sems[slot])

    _start(0, 0)
    for q in range(_NG * _NK):
        g, k = divmod(q, _NK)
        slot = q % 2
        handles[slot].wait()
        if q + 1 < _NG * _NK:
            _start(q + 1, 1 - slot)
        vb = bufs[slot]

        def _row(rr, carry):
            r = g * _GR + rr

            def _pass1(i, c):
                ma, mb, ra, rb = c
                a = vb[rr, pl.ds(i * 32, _LANES)]
                b = vb[rr, pl.ds(i * 32 + _LANES, _LANES)]
                return (jnp.maximum(ma, a), jnp.maximum(mb, b),
                        ra + a, rb + b)

            ma, mb, ra, rb = lax.fori_loop(
                0, _WC // 32, _pass1, (neg_inf, neg_inf, zero, zero),
                unroll=4)
            cm = jnp.maximum(ma, mb)

            m_old = st_m[r]
            m_new = jnp.maximum(m_old, cm)

            def _pass2(i, c):
                sa, sb = c
                a = vb[rr, pl.ds(i * 32, _LANES)]
                b = vb[rr, pl.ds(i * 32 + _LANES, _LANES)]
                return (sa + jnp.exp(a - m_new), sb + jnp.exp(b - m_new))

            sa, sb = lax.fori_loop(0, _WC // 32, _pass2, (zero, zero),
                                   unroll=4)

            st_s[r] = st_s[r] * jnp.exp(m_old - m_new) + (sa + sb)
            st_m[r] = m_new
            st_rs[r] = st_rs[r] + (ra + rb)

            t16 = tl_v[r]
            lo = _C0 + k * _WC
            tsc = t16[0]
            inb = (tsc >= lo) & (tsc < lo + _WC)

            @pl.when(inb)
            def _pick():
                def _p3(i, acc):
                    colv = (lo + i * _LANES) + lane_iota
                    d = vb[rr, pl.ds(i * _LANES, _LANES)]
                    return acc + jnp.where(colv == t16, d, 0.0)
                gadd = lax.fori_loop(0, _WC // _LANES, _p3, zero, unroll=4)
                st_g[r] = st_g[r] + gadd

            return carry

        lax.fori_loop(0, _GR, _row, 0)

    pltpu.sync_copy(st_m, m_out.at[pl.ds(r0, _RPT), :])
    pltpu.sync_copy(st_s, s_out.at[pl.ds(r0, _RPT), :])
    pltpu.sync_copy(st_rs, rs_out.at[pl.ds(r0, _RPT), :])
    pltpu.sync_copy(st_g, g_out.at[pl.ds(r0, _RPT), :])


_sc_stats = functools.partial(
    pl.kernel,
    mesh=plsc.VectorSubcoreMesh(core_axis_name="c", subcore_axis_name="s"),
    out_type=[
        jax.ShapeDtypeStruct((_B, _LANES), jnp.float32),
        jax.ShapeDtypeStruct((_B, _LANES), jnp.float32),
        jax.ShapeDtypeStruct((_B, _LANES), jnp.float32),
        jax.ShapeDtypeStruct((_B, _LANES), jnp.float32),
    ],
    scratch_types=[
        pltpu.VMEM((_GR, _WC), jnp.float32),
        pltpu.VMEM((_GR, _WC), jnp.float32),
        pltpu.VMEM((_RPT, _LANES), jnp.float32),
        pltpu.VMEM((_RPT, _LANES), jnp.float32),
        pltpu.VMEM((_RPT, _LANES), jnp.float32),
        pltpu.VMEM((_RPT, _LANES), jnp.float32),
        pltpu.VMEM((_RPT, _LANES), jnp.int32),
        pltpu.SemaphoreType.DMA,
        pltpu.SemaphoreType.DMA,
    ],
)(_sc_body)


def _merge_kernel(xt_ref, t_ref, mt_ref, st_ref, rst_ref, gt_ref,
                  ms_ref, ss_ref, rss_ref, gs_ref, o_ref):
    mt = mt_ref[...]
    ms16 = ms_ref[...]
    xt = xt_ref[...]
    mtail = jnp.max(xt, axis=1, keepdims=True)
    M = jnp.maximum(jnp.maximum(mt, jnp.max(ms16, axis=1, keepdims=True)),
                    mtail)
    S = (st_ref[...] * jnp.exp(mt - M)
         + jnp.sum(ss_ref[...] * jnp.exp(ms16 - M), axis=1, keepdims=True)
         + jnp.sum(jnp.exp(xt - M), axis=1, keepdims=True))
    rs = (rst_ref[...] + jnp.sum(rss_ref[...], axis=1, keepdims=True)
          + jnp.sum(xt, axis=1, keepdims=True))
    cols = lax.broadcasted_iota(jnp.int32, xt.shape, 1)
    tloc = t_ref[...] - _C1
    g = (gt_ref[...] + jnp.sum(gs_ref[...], axis=1, keepdims=True)
         + jnp.sum(jnp.where(cols == tloc, xt, 0.0), axis=1, keepdims=True))
    lse = M + jnp.log(S)
    eps = _SMOOTHING / (_C - 1)
    rowloss = -eps * (rs - _C * lse) - (_CONF - eps) * (g - lse)
    o_ref[...] = (jnp.sum(rowloss) / _B).reshape(1, 1)


def _merge(xt, t2, mt, st, rst, gt, ms, ss, rss, gs):
    col = pl.BlockSpec((_B, 1), lambda: (0, 0))
    wide = pl.BlockSpec((_B, _LANES), lambda: (0, 0))
    tail = pl.BlockSpec((_B, _TAIL), lambda: (0, 0))
    return pl.pallas_call(
        _merge_kernel,
        in_specs=[tail, col, col, col, col, col, wide, wide, wide, wide],
        out_specs=pl.BlockSpec((1, 1), lambda: (0, 0)),
        out_shape=jax.ShapeDtypeStruct((1, 1), jnp.float32),
    )(xt, t2, mt, st, rst, gt, ms, ss, rss, gs)


def kernel(pred, target):
    t32 = target.astype(jnp.int32)
    t2 = t32.reshape(_B, 1)
    t_rep = jnp.broadcast_to(t2, (_B, _LANES))
    ms, ss, rss, gs = _sc_stats(pred, t_rep)
    mt, st, rst, gt = _tc_partial(pred, t2)
    xt = lax.slice(pred, (0, _C1), (_B, _C))
    out = _merge(xt, t2, mt, st, rst, gt, ms, ss, rss, gs)
    return out[0, 0]
